# Initial kernel scaffold; baseline (speedup 1.0000x reference)
import jax
import jax.numpy as jnp
from jax.experimental import pallas as pl


def kernel(x, hs):
    raise NotImplementedError("write your pallas kernel here")


# speedup vs baseline: 2275.2429x; 2275.2429x over previous
import functools

import jax
import jax.numpy as jnp
import numpy as np
from jax.experimental import pallas as pl

_Q = 128
_RATIO = 2.0
_EPS = 1e-4
_N = 2


@functools.lru_cache(maxsize=None)
def _nco_tables(T: int):
    d = np.float32(_Q / _RATIO + _EPS)
    q = np.float32(_Q)
    acc = np.float32(0.0)
    fl = np.zeros((T, _N), np.int32)
    ce = np.zeros((T, _N), np.int32)
    fr = np.zeros((T, _N), np.float32)
    ok = np.zeros((T, _N), bool)
    for k in range(T):
        a = acc
        for j in range(_N):
            if a < q:
                ind = a
                f = np.float32(np.floor(ind))
                fl[k, j] = int(f)
                ce[k, j] = min(fl[k, j] + 1, _Q - 1)
                fr[k, j] = np.float32(ind - f)
                ok[k, j] = True
                a = np.float32(a + d)
            else:
                ok[k, j] = False
        acc = np.float32(a - q)
    return fl, ce, fr, ok


@functools.lru_cache(maxsize=None)
def _blend_tables(T: int):
    fl, ce, fr, ok = _nco_tables(T)
    assert ok.all(), "NCO produced an inactive output slot for this T"
    out = []
    for j in range(_N):
        lo = int(min(fl[:, j].min(), ce[:, j].min()))
        hi = int(max(fl[:, j].max(), ce[:, j].max()))
        n = hi - lo + 1
        w = np.zeros((n, T), np.float32)
        kk = np.arange(T)
        np.add.at(w, (fl[:, j] - lo, kk), np.float32(1.0) - fr[:, j])
        np.add.at(w, (ce[:, j] - lo, kk), fr[:, j])
        out.append((lo, n, w))
    return tuple(out)


def _fir_blend_body(xp_ref, hs_ref, w0_ref, w1_ref, o_ref, *, taps, r0, n0,
                    r1, n1):
    T = o_ref.shape[1]
    S = jnp.concatenate([xp_ref[0:1, l:l + T] for l in range(taps)], axis=0)
    C0 = jnp.dot(hs_ref[r0:r0 + n0, :], S, preferred_element_type=jnp.float32)
    C1 = jnp.dot(hs_ref[r1:r1 + n1, :], S, preferred_element_type=jnp.float32)
    y0 = w0_ref[0:1, :] * C0[0:1, :]
    for r in range(1, n0):
        y0 = y0 + w0_ref[r:r + 1, :] * C0[r:r + 1, :]
    y1 = w1_ref[0:1, :] * C1[0:1, :]
    for r in range(1, n1):
        y1 = y1 + w1_ref[r:r + 1, :] * C1[r:r + 1, :]
    o_ref[0:1, :] = y0
    o_ref[1:2, :] = y1


def kernel(x, hs):
    T = x.shape[0]
    taps = hs.shape[1]
    (r0, n0, w0), (r1, n1, w1) = _blend_tables(T)
    pad_r = 128 - (taps - 1)
    xp = jnp.pad(x, (taps - 1, pad_r)).reshape(1, T + 128)
    body = functools.partial(_fir_blend_body, taps=taps, r0=r0, n0=n0,
                             r1=r1, n1=n1)
    out2 = pl.pallas_call(
        body,
        out_shape=jax.ShapeDtypeStruct((2, T), jnp.float32),
    )(xp, hs, jnp.asarray(w0), jnp.asarray(w1))
    return out2.T
